# Initial kernel scaffold; baseline (speedup 1.0000x reference)
import jax
import jax.numpy as jnp
from jax.experimental import pallas as pl


def kernel(x, mask, img_ids, Wq, bq, Wk, bk, Wv, bv, avgs, std_devs, noise_x, noise_y):
    raise NotImplementedError("write your pallas kernel here")


# speedup vs baseline: 3.7264x; 3.7264x over previous
import jax
import jax.numpy as jnp
from jax.experimental import pallas as pl
from jax.experimental.pallas import tpu as pltpu

B = 32
S = 576
D = 768
GRID = 24.0


def _fused_kernel(ids_ref, x_ref, gauss_ref, noise_ref, wcat_ref, bcat_ref,
                  out_ref):
    xb = x_ref[0]
    qkv = jax.lax.dot_general(
        xb, wcat_ref[...],
        (((1,), (1,)), ((), ())),
        preferred_element_type=jnp.float32,
    ) + bcat_ref[...]
    q = qkv[:, :D]
    kv = qkv[:, D:]

    mean_x = gauss_ref[0, 0:1, :]
    mean_y = gauss_ref[0, 1:2, :]
    std_x = gauss_ref[0, 2:3, :]
    std_y = gauss_ref[0, 3:4, :]
    nx = noise_ref[0, 0:1, :]
    ny = noise_ref[0, 1:2, :]
    key_x = mean_x + std_x * nx
    key_y = mean_y + std_y * ny
    kx1 = jnp.ceil(key_x)
    kx2 = jnp.floor(key_x)
    ky1 = jnp.ceil(key_y)
    ky2 = jnp.floor(key_y)

    def to_idx(ky, kx):
        idx = GRID * ky + kx
        return jnp.clip(idx, 0.0, float(S - 1)).astype(jnp.int32)

    idxs = (to_idx(ky1, kx1), to_idx(ky1, kx2),
            to_idx(ky2, kx1), to_idx(ky2, kx2))

    rows = jax.lax.broadcasted_iota(jnp.int32, (S, S), 0)
    qks = []
    vs = []
    for idx in idxs:
        oh = (rows == idx).astype(jnp.float32)
        g = jax.lax.dot_general(
            oh, kv,
            (((0,), (0,)), ((), ())),
            preferred_element_type=jnp.float32,
        )
        qks.append(q * g[:, :D])
        vs.append(g[:, D:])

    m = jnp.maximum(jnp.maximum(qks[0], qks[1]), jnp.maximum(qks[2], qks[3]))
    es = [jnp.exp(t - m) for t in qks]
    rden = 1.0 / (es[0] + es[1] + es[2] + es[3])
    for j in range(4):
        out_ref[0, :, j, :] = es[j] * vs[j] * rden


def kernel(x, mask, img_ids, Wq, bq, Wk, bk, Wv, bv, avgs, std_devs,
           noise_x, noise_y):
    del mask
    wcat = jnp.concatenate([Wq, Wk, Wv], axis=0)
    bcat = jnp.concatenate([bq, bk, bv])[None, :]
    gauss = jnp.concatenate([avgs, std_devs], axis=1)
    noise = jnp.stack([noise_x, noise_y], axis=1)

    grid_spec = pltpu.PrefetchScalarGridSpec(
        num_scalar_prefetch=1,
        grid=(B,),
        in_specs=[
            pl.BlockSpec((1, S, D), lambda b, ids: (b, 0, 0)),
            pl.BlockSpec((1, 4, S), lambda b, ids: (ids[b], 0, 0)),
            pl.BlockSpec((1, 2, S), lambda b, ids: (b, 0, 0)),
            pl.BlockSpec((3 * D, D), lambda b, ids: (0, 0)),
            pl.BlockSpec((1, 3 * D), lambda b, ids: (0, 0)),
        ],
        out_specs=pl.BlockSpec((1, S, 4, D), lambda b, ids: (b, 0, 0, 0)),
    )
    return pl.pallas_call(
        _fused_kernel,
        grid_spec=grid_spec,
        out_shape=jax.ShapeDtypeStruct((B, S, 4, D), jnp.float32),
        compiler_params=pltpu.CompilerParams(
            dimension_semantics=("arbitrary",),
        ),
    )(img_ids, x, gauss, noise, wcat, bcat)
